# Initial kernel scaffold; baseline (speedup 1.0000x reference)
import jax
import jax.numpy as jnp
from jax.experimental import pallas as pl


def kernel(sequences, edge_index, edge_attr, node_features, node_indices, Wg, bg, W_ih, W_hh, b_ih, b_hh, fc_w, fc_b):
    raise NotImplementedError("write your pallas kernel here")


# speedup vs baseline: 10.0584x; 10.0584x over previous
import functools
import jax
import jax.numpy as jnp
from jax import lax
from jax.experimental import pallas as pl
from jax.experimental.pallas import tpu as pltpu
from jax.experimental.pallas import tpu_sc as plsc

_NC, _NS = 2, 16
_NW = _NC * _NS
_N = 10000
_NP = 10240
_E = 320000
_EPW = _E // _NW
_RND = 80
_EP = _NW * _RND * 128
_ROWS_PW = _NP // _NS
_DF = 128
_GC = 128
_SF = 32
_T = 20
_H = 64


@functools.partial(
    pl.kernel,
    out_type=jax.ShapeDtypeStruct((_NW, _NP), jnp.float32),
    mesh=plsc.VectorSubcoreMesh(core_axis_name="c", subcore_axis_name="s"),
    scratch_types=[
        pltpu.VMEM((_EPW,), jnp.int32),
        pltpu.VMEM((_NP,), jnp.float32),
    ],
    compiler_params=pltpu.CompilerParams(needs_layout_passes=False),
)
def _deg_kernel(dst_hbm, out_hbm, idx_v, acc_v):
    wid = lax.axis_index("s") * _NC + lax.axis_index("c")

    def zero_body(i, carry):
        acc_v[pl.ds(i * 16, 16)] = jnp.zeros((16,), jnp.float32)
        return carry

    lax.fori_loop(0, _NP // 16, zero_body, 0)

    pltpu.sync_copy(dst_hbm.at[pl.ds(wid * _EPW, _EPW)], idx_v)

    ones16 = jnp.ones((16,), jnp.float32)

    def scat_body(i, carry):
        idx16 = idx_v[pl.ds(i * 16, 16)]
        plsc.addupdate_scatter(acc_v, [idx16], ones16)
        return carry

    lax.fori_loop(0, _EPW // 16, scat_body, 0)
    pltpu.sync_copy(acc_v, out_hbm.at[wid])


@functools.partial(
    pl.kernel,
    out_type=jax.ShapeDtypeStruct((_NC, _NP, _GC), jnp.float32),
    mesh=plsc.VectorSubcoreMesh(core_axis_name="c", subcore_axis_name="s"),
    scratch_types=[
        pltpu.VMEM((_RND, 128), jnp.int32),
        pltpu.VMEM((_RND, 128), jnp.int32),
        pltpu.VMEM((128, _GC), jnp.float32),
        pltpu.VMEM_SHARED((_NP, _GC), jnp.float32),
        pltpu.SemaphoreType.DMA,
    ],
)
def _scat_kernel(src_hbm, dst_hbm, y_hbm, z_hbm, out_hbm,
                 sidx_v, didx_v, rows_v, acc_s, sem):
    c = lax.axis_index("c")
    s = lax.axis_index("s")
    wid = s * _NC + c

    pltpu.sync_copy(z_hbm.at[pl.ds(s * _ROWS_PW, _ROWS_PW)],
                    acc_s.at[pl.ds(s * _ROWS_PW, _ROWS_PW)])
    pltpu.sync_copy(src_hbm.at[pl.ds(wid * _RND, _RND)], sidx_v)
    pltpu.sync_copy(dst_hbm.at[pl.ds(wid * _RND, _RND)], didx_v)
    plsc.subcore_barrier()

    def body(r, carry):
        pltpu.async_copy(y_hbm.at[sidx_v.at[r]], rows_v, sem).wait()
        pltpu.sync_copy(rows_v, acc_s.at[didx_v.at[r]], add=True)
        return carry

    lax.fori_loop(0, _RND, body, 0)
    plsc.subcore_barrier()
    pltpu.sync_copy(acc_s.at[pl.ds(s * _ROWS_PW, _ROWS_PW)],
                    out_hbm.at[c, pl.ds(s * _ROWS_PW, _ROWS_PW)])


def _prep_body(dp_ref, nf_ref, wg_ref, bg_ref, y_ref, sb_ref, di_ref):
    deg = jnp.sum(dp_ref[...], axis=0) + 1.0
    dinv = lax.rsqrt(deg)
    xw = jnp.dot(nf_ref[...], wg_ref[...], preferred_element_type=jnp.float32)
    y_ref[...] = dinv[:, None] * xw
    sb_ref[...] = (dinv * dinv)[:, None] * xw + bg_ref[...]
    di_ref[...] = dinv[:, None]


_prep_call = pl.pallas_call(
    _prep_body,
    grid=(10,),
    in_specs=[
        pl.BlockSpec((_NW, 1024), lambda i: (0, i)),
        pl.BlockSpec((1024, _DF), lambda i: (i, 0)),
        pl.BlockSpec((_DF, _GC), lambda i: (0, 0)),
        pl.BlockSpec((1, _GC), lambda i: (0, 0)),
    ],
    out_specs=[
        pl.BlockSpec((1024, _GC), lambda i: (i, 0)),
        pl.BlockSpec((1024, _GC), lambda i: (i, 0)),
        pl.BlockSpec((1024, 1), lambda i: (i, 0)),
    ],
    out_shape=[
        jax.ShapeDtypeStruct((_NP, _GC), jnp.float32),
        jax.ShapeDtypeStruct((_NP, _GC), jnp.float32),
        jax.ShapeDtypeStruct((_NP, 1), jnp.float32),
    ],
)


def _lstm_body(seq_ref, acc_ref, sb_ref, di_ref,
               wst_ref, wgt_ref, wht_ref, b_ref, fcw_ref, fcb_ref, out_ref):
    acc = acc_ref[0] + acc_ref[1]
    gcn = di_ref[...] * acc + sb_ref[...]
    gbase = jnp.dot(gcn, wgt_ref[...], preferred_element_type=jnp.float32) + b_ref[...]
    nb = gcn.shape[0]
    wst = wst_ref[...]
    wht = wht_ref[...]
    h = jnp.zeros((nb, _H), jnp.float32)
    cc = jnp.zeros((nb, _H), jnp.float32)
    for t in range(_T):
        xt = seq_ref[:, t * _SF:(t + 1) * _SF]
        g = (jnp.dot(xt, wst, preferred_element_type=jnp.float32) + gbase
             + jnp.dot(h, wht, preferred_element_type=jnp.float32))
        ig = jax.nn.sigmoid(g[:, 0:_H])
        fg = jax.nn.sigmoid(g[:, _H:2 * _H])
        gg = jnp.tanh(g[:, 2 * _H:3 * _H])
        og = jax.nn.sigmoid(g[:, 3 * _H:4 * _H])
        cc = fg * cc + ig * gg
        h = og * jnp.tanh(cc)
    out_ref[...] = jnp.dot(h, fcw_ref[...], preferred_element_type=jnp.float32) + fcb_ref[...]


_lstm_call = pl.pallas_call(
    _lstm_body,
    grid=(10,),
    in_specs=[
        pl.BlockSpec((1000, _T * _SF), lambda i: (i, 0)),
        pl.BlockSpec((2, 1000, _GC), lambda i: (0, i, 0)),
        pl.BlockSpec((1000, _GC), lambda i: (i, 0)),
        pl.BlockSpec((1000, 1), lambda i: (i, 0)),
        pl.BlockSpec((_SF, 4 * _H), lambda i: (0, 0)),
        pl.BlockSpec((_GC, 4 * _H), lambda i: (0, 0)),
        pl.BlockSpec((_H, 4 * _H), lambda i: (0, 0)),
        pl.BlockSpec((1, 4 * _H), lambda i: (0, 0)),
        pl.BlockSpec((_H, 1), lambda i: (0, 0)),
        pl.BlockSpec((1, 1), lambda i: (0, 0)),
    ],
    out_specs=pl.BlockSpec((1000, 1), lambda i: (i, 0)),
    out_shape=jax.ShapeDtypeStruct((_N, 1), jnp.float32),
)


def kernel(sequences, edge_index, edge_attr, node_features, node_indices,
           Wg, bg, W_ih, W_hh, b_ih, b_hh, fc_w, fc_b):
    del edge_attr, node_indices
    src = edge_index[0]
    dst = edge_index[1]

    deg_part = _deg_kernel(dst)

    nf_pad = jnp.concatenate(
        [node_features, jnp.zeros((_NP - _N, _DF), jnp.float32)], axis=0)
    y, sbase, dinv = _prep_call(deg_part, nf_pad, Wg, bg.reshape(1, _GC))

    pad = jnp.full((_EP - _E,), _N, jnp.int32)
    src2d = jnp.concatenate([src, pad]).reshape(-1, 128)
    dst2d = jnp.concatenate([dst, pad]).reshape(-1, 128)
    zeros = jnp.zeros((_NP, _GC), jnp.float32)
    acc = _scat_kernel(src2d, dst2d, y, zeros)

    seq2d = sequences.reshape(_N, _T * _SF)
    w_ih_t = W_ih.T
    out = _lstm_call(
        seq2d, acc, sbase, dinv,
        w_ih_t[:_SF], w_ih_t[_SF:], W_hh.T,
        (b_ih + b_hh).reshape(1, 4 * _H),
        fc_w.T, fc_b.reshape(1, 1),
    )
    return out


# speedup vs baseline: 10.7088x; 1.0647x over previous
import functools
import jax
import jax.numpy as jnp
from jax import lax
from jax.experimental import pallas as pl
from jax.experimental.pallas import tpu as pltpu
from jax.experimental.pallas import tpu_sc as plsc

_NC, _NS = 2, 16
_NW = _NC * _NS
_N = 10000
_NP = 10240
_E = 320000
_EPW = _E // _NW
_RND = 80
_GRP = 16
_EP = _NW * _RND * 128
_ROWS_PW = _NP // _NS
_DF = 128
_GC = 128
_SF = 32
_T = 20
_H = 64


@functools.partial(
    pl.kernel,
    out_type=jax.ShapeDtypeStruct((_NW, _NP), jnp.float32),
    mesh=plsc.VectorSubcoreMesh(core_axis_name="c", subcore_axis_name="s"),
    scratch_types=[
        pltpu.VMEM((_EPW,), jnp.int32),
        pltpu.VMEM((_NP,), jnp.float32),
    ],
    compiler_params=pltpu.CompilerParams(needs_layout_passes=False),
)
def _deg_kernel(dst_hbm, out_hbm, idx_v, acc_v):
    wid = lax.axis_index("s") * _NC + lax.axis_index("c")

    def zero_body(i, carry):
        acc_v[pl.ds(i * 16, 16)] = jnp.zeros((16,), jnp.float32)
        return carry

    lax.fori_loop(0, _NP // 16, zero_body, 0)

    pltpu.sync_copy(dst_hbm.at[pl.ds(wid * _EPW, _EPW)], idx_v)

    ones16 = jnp.ones((16,), jnp.float32)

    def scat_body(i, carry):
        idx16 = idx_v[pl.ds(i * 16, 16)]
        plsc.addupdate_scatter(acc_v, [idx16], ones16)
        return carry

    lax.fori_loop(0, _EPW // 16, scat_body, 0)
    pltpu.sync_copy(acc_v, out_hbm.at[wid])


@functools.partial(
    pl.kernel,
    out_type=jax.ShapeDtypeStruct((_NC, _NP, _GC), jnp.float32),
    mesh=plsc.VectorSubcoreMesh(core_axis_name="c", subcore_axis_name="s"),
    scratch_types=[
        pltpu.VMEM((2, _GRP, 128), jnp.int32),
        pltpu.VMEM((2, _GRP, 128), jnp.int32),
        pltpu.VMEM((2, 128, _GC), jnp.float32),
        pltpu.VMEM_SHARED((_NP, _GC), jnp.float32),
        pltpu.SemaphoreType.DMA,
        pltpu.SemaphoreType.DMA,
        pltpu.SemaphoreType.DMA,
    ],
)
def _scat_kernel(src_hbm, dst_hbm, y_hbm, z_hbm, out_hbm,
                 sidx_v, didx_v, rows_v, acc_s, s0, s1, isem):
    c = lax.axis_index("c")
    s = lax.axis_index("s")
    wid = s * _NC + c
    sems = (s0, s1)
    ngrp = _RND // _GRP

    pltpu.sync_copy(z_hbm.at[pl.ds(s * _ROWS_PW, _ROWS_PW)],
                    acc_s.at[pl.ds(s * _ROWS_PW, _ROWS_PW)])
    base0 = wid * _RND
    pltpu.async_copy(src_hbm.at[pl.ds(base0, _GRP)], sidx_v.at[0], isem)
    pltpu.async_copy(dst_hbm.at[pl.ds(base0, _GRP)], didx_v.at[0], isem)
    plsc.subcore_barrier()

    def group(gi, carry):
        par = lax.rem(gi, 2)
        base = wid * _RND + gi * _GRP
        pltpu.make_async_copy(
            src_hbm.at[pl.ds(base, _GRP)], sidx_v.at[par], isem).wait()
        pltpu.make_async_copy(
            dst_hbm.at[pl.ds(base, _GRP)], didx_v.at[par], isem).wait()
        nbase = wid * _RND + jnp.minimum(gi + 1, ngrp - 1) * _GRP
        pltpu.async_copy(src_hbm.at[pl.ds(nbase, _GRP)],
                         sidx_v.at[1 - par], isem)
        pltpu.async_copy(dst_hbm.at[pl.ds(nbase, _GRP)],
                         didx_v.at[1 - par], isem)
        sidx = sidx_v.at[par]
        didx = didx_v.at[par]
        for b in range(2):
            pltpu.async_copy(y_hbm.at[sidx.at[b]], rows_v.at[b], sems[b])
        for b in range(_GRP):
            p = b % 2
            pltpu.make_async_copy(
                y_hbm.at[sidx.at[b]], rows_v.at[p], sems[p]).wait()
            pltpu.sync_copy(rows_v.at[p], acc_s.at[didx.at[b]], add=True)
            if b + 2 < _GRP:
                pltpu.async_copy(
                    y_hbm.at[sidx.at[b + 2]], rows_v.at[p], sems[p])
        return carry

    lax.fori_loop(0, ngrp, group, 0)
    lastb = wid * _RND + (ngrp - 1) * _GRP
    pltpu.make_async_copy(
        src_hbm.at[pl.ds(lastb, _GRP)], sidx_v.at[1], isem).wait()
    pltpu.make_async_copy(
        dst_hbm.at[pl.ds(lastb, _GRP)], didx_v.at[1], isem).wait()
    plsc.subcore_barrier()
    pltpu.sync_copy(acc_s.at[pl.ds(s * _ROWS_PW, _ROWS_PW)],
                    out_hbm.at[c, pl.ds(s * _ROWS_PW, _ROWS_PW)])


def _prep_body(dp_ref, nf_ref, wg_ref, bg_ref, y_ref, sb_ref, di_ref):
    deg = jnp.sum(dp_ref[...], axis=0) + 1.0
    dinv = lax.rsqrt(deg)
    xw = jnp.dot(nf_ref[...], wg_ref[...], preferred_element_type=jnp.float32)
    y_ref[...] = dinv[:, None] * xw
    sb_ref[...] = (dinv * dinv)[:, None] * xw + bg_ref[...]
    di_ref[...] = dinv[:, None]


_prep_call = pl.pallas_call(
    _prep_body,
    grid=(10,),
    in_specs=[
        pl.BlockSpec((_NW, 1024), lambda i: (0, i)),
        pl.BlockSpec((1024, _DF), lambda i: (i, 0)),
        pl.BlockSpec((_DF, _GC), lambda i: (0, 0)),
        pl.BlockSpec((1, _GC), lambda i: (0, 0)),
    ],
    out_specs=[
        pl.BlockSpec((1024, _GC), lambda i: (i, 0)),
        pl.BlockSpec((1024, _GC), lambda i: (i, 0)),
        pl.BlockSpec((1024, 1), lambda i: (i, 0)),
    ],
    out_shape=[
        jax.ShapeDtypeStruct((_NP, _GC), jnp.float32),
        jax.ShapeDtypeStruct((_NP, _GC), jnp.float32),
        jax.ShapeDtypeStruct((_NP, 1), jnp.float32),
    ],
)


def _lstm_body(seq_ref, acc_ref, sb_ref, di_ref,
               wst_ref, wgt_ref, wht_ref, b_ref, fcw_ref, fcb_ref, out_ref):
    acc = acc_ref[0] + acc_ref[1]
    gcn = di_ref[...] * acc + sb_ref[...]
    gbase = jnp.dot(gcn, wgt_ref[...], preferred_element_type=jnp.float32) + b_ref[...]
    nb = gcn.shape[0]
    wst = wst_ref[...]
    wht = wht_ref[...]
    h = jnp.zeros((nb, _H), jnp.float32)
    cc = jnp.zeros((nb, _H), jnp.float32)
    for t in range(_T):
        xt = seq_ref[:, t * _SF:(t + 1) * _SF]
        g = (jnp.dot(xt, wst, preferred_element_type=jnp.float32) + gbase
             + jnp.dot(h, wht, preferred_element_type=jnp.float32))
        ig = jax.nn.sigmoid(g[:, 0:_H])
        fg = jax.nn.sigmoid(g[:, _H:2 * _H])
        gg = jnp.tanh(g[:, 2 * _H:3 * _H])
        og = jax.nn.sigmoid(g[:, 3 * _H:4 * _H])
        cc = fg * cc + ig * gg
        h = og * jnp.tanh(cc)
    out_ref[...] = jnp.dot(h, fcw_ref[...], preferred_element_type=jnp.float32) + fcb_ref[...]


_lstm_call = pl.pallas_call(
    _lstm_body,
    grid=(10,),
    in_specs=[
        pl.BlockSpec((1000, _T * _SF), lambda i: (i, 0)),
        pl.BlockSpec((2, 1000, _GC), lambda i: (0, i, 0)),
        pl.BlockSpec((1000, _GC), lambda i: (i, 0)),
        pl.BlockSpec((1000, 1), lambda i: (i, 0)),
        pl.BlockSpec((_SF, 4 * _H), lambda i: (0, 0)),
        pl.BlockSpec((_GC, 4 * _H), lambda i: (0, 0)),
        pl.BlockSpec((_H, 4 * _H), lambda i: (0, 0)),
        pl.BlockSpec((1, 4 * _H), lambda i: (0, 0)),
        pl.BlockSpec((_H, 1), lambda i: (0, 0)),
        pl.BlockSpec((1, 1), lambda i: (0, 0)),
    ],
    out_specs=pl.BlockSpec((1000, 1), lambda i: (i, 0)),
    out_shape=jax.ShapeDtypeStruct((_N, 1), jnp.float32),
)


def kernel(sequences, edge_index, edge_attr, node_features, node_indices,
           Wg, bg, W_ih, W_hh, b_ih, b_hh, fc_w, fc_b):
    del edge_attr, node_indices
    src = edge_index[0]
    dst = edge_index[1]

    deg_part = _deg_kernel(dst)

    nf_pad = jnp.concatenate(
        [node_features, jnp.zeros((_NP - _N, _DF), jnp.float32)], axis=0)
    y, sbase, dinv = _prep_call(deg_part, nf_pad, Wg, bg.reshape(1, _GC))

    pad = jnp.full((_EP - _E,), _N, jnp.int32)
    src2d = jnp.concatenate([src, pad]).reshape(-1, 128)
    dst2d = jnp.concatenate([dst, pad]).reshape(-1, 128)
    zeros = jnp.zeros((_NP, _GC), jnp.float32)
    acc = _scat_kernel(src2d, dst2d, y, zeros)

    seq2d = sequences.reshape(_N, _T * _SF)
    w_ih_t = W_ih.T
    out = _lstm_call(
        seq2d, acc, sbase, dinv,
        w_ih_t[:_SF], w_ih_t[_SF:], W_hh.T,
        (b_ih + b_hh).reshape(1, 4 * _H),
        fc_w.T, fc_b.reshape(1, 1),
    )
    return out
